# baseline (speedup 1.0000x reference)
import functools

import jax
import jax.numpy as jnp
from jax import lax
from jax.experimental import pallas as pl
from jax.experimental.pallas import tpu as pltpu
from jax.experimental.pallas import tpu_sc as plsc

N = 10000
E = 320000
H = 8
D = 16
OUT = H * D

_BN = 2000
_BE = 2000

_NS = 16
_B = 80
_EPT = E // _NS
_FULL = _EPT // _B
_NPAIR = _FULL // 2
_RPT = N // _NS
_RCHUNKS = [(i * _B, _B) for i in range(_RPT // _B)] + [
    (_RPT - _RPT % _B, _RPT % _B)]


def _ln(y, g, b):
    mu = jnp.mean(y, axis=-1, keepdims=True)
    yc = y - mu
    var = jnp.mean(yc * yc, axis=-1, keepdims=True)
    return yc * lax.rsqrt(var + 1e-5) * g + b


def _qkv_body(temp_ref, h_ref, wq_ref, bq_ref, wk_ref, bk_ref, wv_ref, bv_ref,
              g_ref, b_ref, q_out, vk_out):
    x = h_ref[...]
    g = g_ref[...]
    b = b_ref[...]
    inv = 1.0 / (4.0 * temp_ref[0])
    q = _ln(jnp.dot(x, wq_ref[...], preferred_element_type=jnp.float32)
            + bq_ref[...], g, b) * inv
    k = _ln(jnp.dot(x, wk_ref[...], preferred_element_type=jnp.float32)
            + bk_ref[...], g, b)
    v = _ln(jnp.dot(x, wv_ref[...], preferred_element_type=jnp.float32)
            + bv_ref[...], g, b)
    q_out[0] = q[:, :64]
    q_out[1] = q[:, 64:]
    vk_out[0] = jnp.concatenate([v[:, :64], k[:, :64]], axis=1)
    vk_out[1] = jnp.concatenate([v[:, 64:], k[:, 64:]], axis=1)


def _qkv_call(temperature, h, Wq, bq, Wk, bk, Wv, bv, g, b):
    wspec = pl.BlockSpec((OUT, OUT), lambda i: (0, 0))
    bspec = pl.BlockSpec((OUT,), lambda i: (0,))
    return pl.pallas_call(
        _qkv_body,
        grid=(N // _BN,),
        in_specs=[
            pl.BlockSpec(memory_space=pltpu.SMEM),
            pl.BlockSpec((_BN, OUT), lambda i: (i, 0)),
            wspec, bspec, wspec, bspec, wspec, bspec, bspec, bspec,
        ],
        out_specs=[pl.BlockSpec((2, _BN, 64), lambda i: (0, i, 0)),
                   pl.BlockSpec((2, _BN, 128), lambda i: (0, i, 0))],
        out_shape=[jax.ShapeDtypeStruct((2, N, 64), jnp.float32),
                   jax.ShapeDtypeStruct((2, N, 128), jnp.float32)],
    )(temperature, h, Wq, bq, Wk, bk, Wv, bv, g, b)


def _base_body(e_ref, sp_ref, we_ref, be_ref, wp_ref, bp_ref, pos_ref,
               g_ref, b_ref, out_ref):
    g = g_ref[...]
    b = b_ref[...]
    pe = _ln(jnp.dot(e_ref[...], we_ref[...], preferred_element_type=jnp.float32)
             + be_ref[...], g, b)
    lp = _ln(jnp.dot(sp_ref[...], wp_ref[...], preferred_element_type=jnp.float32)
             + bp_ref[...] + pos_ref[...], g, b)
    out_ref[...] = pe + lp


def _base_call(e, sp, We, be, Wp, bp, pos, g, b):
    wspec = pl.BlockSpec((OUT, OUT), lambda i: (0, 0))
    bspec = pl.BlockSpec((OUT,), lambda i: (0,))
    return pl.pallas_call(
        _base_body,
        grid=(E // _BE,),
        in_specs=[
            pl.BlockSpec((_BE, OUT), lambda i: (i, 0)),
            pl.BlockSpec((_BE, OUT), lambda i: (i, 0)),
            wspec, bspec, wspec, bspec,
            pl.BlockSpec((1, OUT), lambda i: (0, 0)),
            bspec, bspec,
        ],
        out_specs=pl.BlockSpec((_BE, OUT), lambda i: (i, 0)),
        out_shape=jax.ShapeDtypeStruct((E, OUT), jnp.float32),
    )(e, sp, We, be, Wp, bp, pos, g, b)


_MESH = plsc.VectorSubcoreMesh(core_axis_name="c", subcore_axis_name="s")


@functools.partial(
    pl.kernel,
    out_type=[
        jax.ShapeDtypeStruct((N, OUT), jnp.float32),
        jax.ShapeDtypeStruct((E, OUT), jnp.float32),
    ],
    mesh=_MESH,
    compiler_params=pltpu.CompilerParams(use_tc_tiling_on_sc=False,
                                          needs_layout_passes=False),
    scratch_types=[
        pltpu.VMEM((2, _B), jnp.int32),
        pltpu.VMEM((_B,), jnp.int32),
        pltpu.VMEM((_B,), jnp.int32),
        pltpu.VMEM((_B, 128), jnp.float32),
        pltpu.VMEM((_B, 64), jnp.float32),
        pltpu.VMEM((_B, 64), jnp.float32),
        pltpu.VMEM((2, _B), jnp.int32),
        pltpu.VMEM((_B,), jnp.int32),
        pltpu.VMEM((_B,), jnp.int32),
        pltpu.VMEM((_B, 128), jnp.float32),
        pltpu.VMEM((_B, 64), jnp.float32),
        pltpu.VMEM((_B, 64), jnp.float32),
        pltpu.VMEM_SHARED((N, 128), jnp.float32),
        pltpu.SemaphoreType.DMA,
        pltpu.SemaphoreType.DMA,
        pltpu.SemaphoreType.DMA,
        pltpu.SemaphoreType.DMA,
    ],
)
def _sc_attn(q_hbm, vk_hbm, base_hbm, ei_hbm,
             hout_hbm, eout_hbm,
             idx0, gsrc0, gdst0, vkbuf0, qbuf0, bbuf0,
             idx1, gsrc1, gdst1, vkbuf1, qbuf1, bbuf1,
             acc_sh, semg0, semg1, semw0, semw1):
    c = lax.axis_index("c")
    s = lax.axis_index("s")
    cN = c * N
    col0 = c * 64
    ebase = s * _EPT

    sets = [
        (idx0, gsrc0, gdst0, vkbuf0, qbuf0, bbuf0, semg0, semw0),
        (idx1, gsrc1, gdst1, vkbuf1, qbuf1, bbuf1, semg1, semw1),
    ]

    def _drain_eout(si, off):
        _, _, _, _, _, bbuf, _, semw = sets[si]
        pltpu.make_async_copy(bbuf, eout_hbm.at[pl.ds(off, _B),
                                                pl.ds(col0, 64)], semw).wait()

    def _prefetch(si, off, drain):
        idx, gsrc, gdst, vkbuf, qbuf, bbuf, semg, _ = sets[si]
        pltpu.async_copy(ei_hbm.at[:, pl.ds(off, _B)], idx, semg).wait()

        def _bias(i, _):
            o = i * 16
            gsrc[pl.ds(o, 16)] = idx[0, pl.ds(o, 16)] + cN
            gdst[pl.ds(o, 16)] = idx[1, pl.ds(o, 16)] + cN
            return 0

        lax.fori_loop(0, _B // 16, _bias, 0)

        pltpu.async_copy(vk_hbm.at[gsrc], vkbuf, semg)
        pltpu.async_copy(q_hbm.at[gdst], qbuf, semg)
        if drain:
            _drain_eout(si, off - 2 * _B)
        pltpu.async_copy(base_hbm.at[pl.ds(off, _B), pl.ds(col0, 64)],
                         bbuf, semg)

    def _wait_gathers(si, off):
        idx, gsrc, gdst, vkbuf, qbuf, bbuf, semg, _ = sets[si]
        pltpu.make_async_copy(vk_hbm.at[gsrc], vkbuf, semg).wait()
        pltpu.make_async_copy(q_hbm.at[gdst], qbuf, semg).wait()
        pltpu.make_async_copy(base_hbm.at[pl.ds(off, _B), pl.ds(col0, 64)],
                              bbuf, semg).wait()

    def _compute(si):
        _, _, _, vkbuf, qbuf, bbuf, _, _ = sets[si]

        def _one(i):
            for hh in range(4):
                sl = pl.ds(hh * 16, 16)
                sk = pl.ds(64 + hh * 16, 16)
                sc = vkbuf[i, sk] * qbuf[i, sl] + bbuf[i, sl]
                sc = jnp.minimum(jnp.maximum(sc, -5.0), 5.0)
                p = jnp.exp(sc)
                r = p / jnp.sum(p)
                bbuf[i, sl] = r
                vkbuf[i, sk] = r
                vkbuf[i, sl] = vkbuf[i, sl] * r

        def _edge(i, _):
            _one(2 * i)
            _one(2 * i + 1)
            return 0

        lax.fori_loop(0, _B // 2, _edge, 0)

    def _writes(si, off):
        idx, _, _, vkbuf, _, bbuf, _, semw = sets[si]
        pltpu.async_copy(bbuf, eout_hbm.at[pl.ds(off, _B), pl.ds(col0, 64)],
                         semw)
        pltpu.sync_copy(vkbuf, acc_sh.at[idx.at[1]], add=True)

    zv = jnp.zeros((16,), jnp.float32)

    def _zb(i, _):
        for j in range(8):
            vkbuf0[i, pl.ds(j * 16, 16)] = zv
        return 0

    lax.fori_loop(0, _B, _zb, 0)
    for (ro, sz) in _RCHUNKS:
        r0 = s * _RPT + ro
        pltpu.sync_copy(vkbuf0.at[pl.ds(0, sz)], acc_sh.at[pl.ds(r0, sz)])
    plsc.subcore_barrier()

    _prefetch(0, ebase, drain=False)
    _prefetch(1, ebase + _B, drain=False)

    def _pair(j2, _):
        off0 = ebase + (2 * j2) * _B
        off1 = off0 + _B
        _wait_gathers(0, off0)
        _compute(0)
        _writes(0, off0)
        _prefetch(0, off0 + 2 * _B, drain=True)

        _wait_gathers(1, off1)
        _compute(1)
        _writes(1, off1)
        _prefetch(1, off1 + 2 * _B, drain=True)
        return 0

    lax.fori_loop(0, _NPAIR - 1, _pair, 0)
    last0 = ebase + (_FULL - 2) * _B
    for si, off in ((0, last0), (1, last0 + _B)):
        _wait_gathers(si, off)
        _compute(si)
        _writes(si, off)
        _drain_eout(si, off)

    plsc.subcore_barrier()

    for (ro, sz) in _RCHUNKS:
        r0 = s * _RPT + ro
        pltpu.sync_copy(acc_sh.at[pl.ds(r0, sz)], vkbuf0.at[pl.ds(0, sz)])

        def _dv(i, _):
            for hh in range(4):
                qbuf0[i, pl.ds(hh * 16, 16)] = (
                    vkbuf0[i, pl.ds(hh * 16, 16)]
                    / (vkbuf0[i, pl.ds(64 + hh * 16, 16)] + 1e-6))
            return 0

        lax.fori_loop(0, sz, _dv, 0)
        pltpu.sync_copy(qbuf0.at[pl.ds(0, sz)],
                        hout_hbm.at[pl.ds(r0, sz), pl.ds(col0, 64)])


def kernel(h, e, spatial_pos, edge_index, Wq, bq, Wk, bk, Wv, bv, We, be,
           Wp, bp, ln_g, ln_b, pos_embedding, temperature):
    qs, vks = _qkv_call(temperature, h, Wq, bq, Wk, bk, Wv, bv, ln_g, ln_b)
    base = _base_call(e, spatial_pos, We, be, Wp, bp, pos_embedding, ln_g, ln_b)
    h_out, e_out = _sc_attn(
        qs.reshape(2 * N, 64), vks.reshape(2 * N, 128), base, edge_index)
    return h_out.reshape(N, H, D), e_out.reshape(E, H, D)
